# baseline (speedup 1.0000x reference)
import functools

import jax
import jax.numpy as jnp
from jax import lax
from jax.experimental import pallas as pl
from jax.experimental.pallas import tpu as pltpu
from jax.experimental.pallas import tpu_sc as plsc

N = 10000
E = 160000
NW = 32
CH = 128
NGCH = (2 * E) // CH
GPW = 80
GPAD = NW * GPW * CH
NSCH = E // CH
SPW = 40
EPAD = NW * SPW * CH
TRASH = N
ZCH = 40
NZCH = N // ZCH
NBUF = 4
SNBUF = 2

_mesh = plsc.VectorSubcoreMesh(core_axis_name="c", subcore_axis_name="s")


def _sc_gather(table, gidx, gpw):
    @functools.partial(
        pl.kernel,
        out_type=jax.ShapeDtypeStruct((NW * gpw * CH, 128), jnp.float32),
        mesh=_mesh,
        scratch_types=[
            pltpu.VMEM((gpw, CH), jnp.int32),
            pltpu.VMEM((NBUF, CH, 128), jnp.float32),
        ] + [pltpu.SemaphoreType.DMA] * (2 * NBUF),
    )
    def k(tab_hbm, idx_hbm, out_hbm, idx_v, rows_v, *sems):
        gsems, wsems = sems[:NBUF], sems[NBUF:]
        cid = lax.axis_index("c")
        sid = lax.axis_index("s")
        w = cid * 16 + sid
        pltpu.sync_copy(idx_hbm.at[w], idx_v)

        @pl.loop(0, gpw, step=NBUF)
        def _(j):
            hg = [pltpu.async_copy(tab_hbm.at[idx_v.at[j + b]],
                                   rows_v.at[b], gsems[b])
                  for b in range(NBUF)]
            hw = []
            for b in range(NBUF):
                hg[b].wait()
                c = (w * gpw + j + b) * CH
                hw.append(pltpu.async_copy(rows_v.at[b],
                                           out_hbm.at[pl.ds(c, CH)], wsems[b]))
            for b in range(NBUF):
                hw[b].wait()

    return k(table, gidx)


def _sc_segment_sum(e_rows, dst2, spw):
    @functools.partial(
        pl.kernel,
        out_type=jax.ShapeDtypeStruct((2, N, 128), jnp.float32),
        mesh=_mesh,
        scratch_types=[
            pltpu.VMEM((spw, CH), jnp.int32),
            pltpu.VMEM((SNBUF, CH, 128), jnp.float32),
            pltpu.VMEM((ZCH, 128), jnp.float32),
            pltpu.VMEM_SHARED((N + 8, 128), jnp.float32),
        ] + [pltpu.SemaphoreType.DMA] * SNBUF,
    )
    def k(e_hbm, idx_hbm, p_hbm, idx_v, rows_v, zbuf, acc, *sems):
        cid = lax.axis_index("c")
        sid = lax.axis_index("s")
        w = cid * 16 + sid

        @pl.loop(0, ZCH)
        def _(i):
            @pl.loop(0, 8)
            def _(l):
                zbuf[i, pl.ds(l * 16, 16)] = jnp.zeros((16,), jnp.float32)

        @pl.loop(0, NZCH // 16 + 1)
        def _(i):
            c = i * 16 + sid

            @pl.when(c < NZCH)
            def _():
                pltpu.sync_copy(zbuf, acc.at[pl.ds(c * ZCH, ZCH)])

        plsc.subcore_barrier()

        pltpu.sync_copy(idx_hbm.at[w], idx_v)

        @pl.loop(0, spw, step=SNBUF)
        def _(j):
            hl = [pltpu.async_copy(
                      e_hbm.at[pl.ds((w * spw + j + b) * CH, CH)],
                      rows_v.at[b], sems[b])
                  for b in range(SNBUF)]
            for b in range(SNBUF):
                hl[b].wait()
                pltpu.sync_copy(rows_v.at[b], acc.at[idx_v.at[j + b]],
                                add=True)

        plsc.subcore_barrier()

        @pl.loop(0, NZCH // 16 + 1)
        def _(i):
            c = i * 16 + sid

            @pl.when(c < NZCH)
            def _():
                pltpu.sync_copy(acc.at[pl.ds(c * ZCH, ZCH)],
                                p_hbm.at[cid, pl.ds(c * ZCH, ZCH)])

    return k(e_rows, dst2)


def _dot(x, w):
    return jnp.dot(x, w, preferred_element_type=jnp.float32)


def _wspec(shape):
    return pl.BlockSpec(shape, lambda i: (0,) * len(shape))


def _mlp3_body(x_ref, w1, b1, w2, b2, w3, b3, o_ref):
    h = jax.nn.relu(_dot(x_ref[...], w1[...]) + b1[...])
    h = jax.nn.relu(_dot(h, w2[...]) + b2[...])
    o_ref[...] = _dot(h, w3[...]) + b3[...]


def _mlp3(x, p, bm, out_rows=None):
    (w1, b1), (w2, b2), (w3, b3) = p
    m, din = x.shape
    dout = w3.shape[1]
    return pl.pallas_call(
        _mlp3_body,
        grid=(m // bm,),
        in_specs=[
            pl.BlockSpec((bm, din), lambda i: (i, 0)),
            _wspec(w1.shape), _wspec((1, w1.shape[1])),
            _wspec(w2.shape), _wspec((1, w2.shape[1])),
            _wspec(w3.shape), _wspec((1, w3.shape[1])),
        ],
        out_specs=pl.BlockSpec((bm, dout), lambda i: (i, 0)),
        out_shape=jax.ShapeDtypeStruct((out_rows or m, dout), jnp.float32),
    )(x, w1, b1.reshape(1, -1), w2, b2.reshape(1, -1), w3, b3.reshape(1, -1))


def _edge_mlp_body(e_ref, gs_ref, gd_ref, w1e, w1a, w1b, b1, w2, b2, w3, b3,
                   o_ref):
    h = (_dot(e_ref[...], w1e[...]) + _dot(gs_ref[...], w1a[...])
         + _dot(gd_ref[...], w1b[...]))
    h = jax.nn.relu(h + b1[...])
    h = jax.nn.relu(_dot(h, w2[...]) + b2[...])
    h = _dot(h, w3[...]) + b3[...]
    o_ref[...] = (h + e_ref[...]) * 0.5


def _edge_mlp(e, g, p, bm, rows, out_pad, eoff=0):
    (w1, b1), (w2, b2), (w3, b3) = p
    w1e, w1a, w1b = w1[:128], w1[128:256], w1[256:]
    goff = rows // bm
    return pl.pallas_call(
        _edge_mlp_body,
        grid=(rows // bm,),
        in_specs=[
            pl.BlockSpec((bm, 128), lambda i: (i + eoff, 0)),
            pl.BlockSpec((bm, 128), lambda i: (i, 0)),
            pl.BlockSpec((bm, 128), lambda i: (i + goff, 0)),
            _wspec((128, 256)), _wspec((128, 256)), _wspec((128, 256)),
            _wspec((1, 256)),
            _wspec((256, 256)), _wspec((1, 256)),
            _wspec((256, 128)), _wspec((1, 128)),
        ],
        out_specs=pl.BlockSpec((bm, 128), lambda i: (i, 0)),
        out_shape=jax.ShapeDtypeStruct((out_pad, 128), jnp.float32),
    )(e, g, g, w1e, w1a, w1b, b1.reshape(1, -1), w2, b2.reshape(1, -1), w3,
      b3.reshape(1, -1))


def _node_mlp_body(n_ref, pa_ref, pb_ref, w1n, w1s, b1, w2, b2, w3, b3,
                   o_ref):
    s = pa_ref[0] + pa_ref[1] + pb_ref[0] + pb_ref[1]
    h = _dot(n_ref[...], w1n[...]) + _dot(s, w1s[...])
    h = jax.nn.relu(h + b1[...])
    h = jax.nn.relu(_dot(h, w2[...]) + b2[...])
    h = _dot(h, w3[...]) + b3[...]
    o_ref[...] = (h + n_ref[...]) * 0.5


def _node_mlp(n_out, pa, pb, p, bm):
    (w1, b1), (w2, b2), (w3, b3) = p
    w1n, w1s = w1[:128], w1[128:]
    return pl.pallas_call(
        _node_mlp_body,
        grid=(N // bm,),
        in_specs=[
            pl.BlockSpec((bm, 128), lambda i: (i, 0)),
            pl.BlockSpec((2, bm, 128), lambda i: (0, i, 0)),
            pl.BlockSpec((2, bm, 128), lambda i: (0, i, 0)),
            _wspec((128, 256)), _wspec((128, 256)), _wspec((1, 256)),
            _wspec((256, 256)), _wspec((1, 256)),
            _wspec((256, 128)), _wspec((1, 128)),
        ],
        out_specs=pl.BlockSpec((bm, 128), lambda i: (i, 0)),
        out_shape=jax.ShapeDtypeStruct((N, 128), jnp.float32),
    )(n_out, pa, pb, w1n, w1s, b1.reshape(1, -1), w2, b2.reshape(1, -1),
      w3, b3.reshape(1, -1))


EH = E // 2
GPW_H = 40
SPW_H = 20
EPAD_H = NW * SPW_H * CH


def kernel(edges, n_feats, e_feats, n_feats_const, params):
    ed = edges[0]
    nf = jnp.concatenate([n_feats[0], n_feats_const[0]], axis=-1)
    ef = e_feats[0]

    gidx, dsts = [], []
    for h in range(2):
        edh = ed[h * EH:(h + 1) * EH]
        gp = (jnp.arange(NW * GPW_H * CH - 2 * EH, dtype=jnp.int32) * 1237) % N
        gidx.append(jnp.concatenate(
            [edh[:, 0], edh[:, 1], gp]).reshape(NW, GPW_H, CH))
        sp = TRASH + (jnp.arange(EPAD_H - EH, dtype=jnp.int32) % 8)
        dsts.append(jnp.concatenate([edh[:, 0], sp]).reshape(NW, SPW_H, CH))

    n_out = _mlp3(nf, params['n_encod'], 2000)
    eh = [_mlp3(ef[h * EH:(h + 1) * EH], params['e_encod'], 4000, EPAD_H)
          for h in range(2)]

    for e_p, n_p in zip(params['e_proc'], params['n_proc']):
        g = [_sc_gather(n_out, gidx[h], GPW_H) for h in range(2)]
        eh = [_edge_mlp(eh[h], g[h], e_p, 2000, EH, EPAD_H)
              for h in range(2)]
        pa = _sc_segment_sum(eh[0], dsts[0], SPW_H)
        pb = _sc_segment_sum(eh[1], dsts[1], SPW_H)
        n_out = _node_mlp(n_out, pa, pb, n_p, 2000)

    out = _mlp3(n_out, params['decod'], 2000)
    return out[None]


# speedup vs baseline: 1.0345x; 1.0345x over previous
import functools

import jax
import jax.numpy as jnp
from jax import lax
from jax.experimental import pallas as pl
from jax.experimental.pallas import tpu as pltpu
from jax.experimental.pallas import tpu_sc as plsc

N = 10000
E = 160000
NW = 32
CH = 128
NGCH = (2 * E) // CH
GPW = 80
GPAD = NW * GPW * CH
NSCH = E // CH
SPW = 40
EPAD = NW * SPW * CH
TRASH = N
ZCH = 40
NZCH = N // ZCH
NBUF = 4
SNBUF = 2

_mesh = plsc.VectorSubcoreMesh(core_axis_name="c", subcore_axis_name="s")


def _sc_gather(table, gidx, gpw):
    @functools.partial(
        pl.kernel,
        out_type=jax.ShapeDtypeStruct((NW * gpw * CH, 128), jnp.float32),
        mesh=_mesh,
        scratch_types=[
            pltpu.VMEM((gpw, CH), jnp.int32),
            pltpu.VMEM((NBUF, CH, 128), jnp.float32),
        ] + [pltpu.SemaphoreType.DMA] * (2 * NBUF),
    )
    def k(tab_hbm, idx_hbm, out_hbm, idx_v, rows_v, *sems):
        gsems, wsems = sems[:NBUF], sems[NBUF:]
        cid = lax.axis_index("c")
        sid = lax.axis_index("s")
        w = cid * 16 + sid
        pltpu.sync_copy(idx_hbm.at[w], idx_v)

        @pl.loop(0, gpw, step=NBUF)
        def _(j):
            hg = [pltpu.async_copy(tab_hbm.at[idx_v.at[j + b]],
                                   rows_v.at[b], gsems[b])
                  for b in range(NBUF)]
            hw = []
            for b in range(NBUF):
                hg[b].wait()
                c = (w * gpw + j + b) * CH
                hw.append(pltpu.async_copy(rows_v.at[b],
                                           out_hbm.at[pl.ds(c, CH)], wsems[b]))
            for b in range(NBUF):
                hw[b].wait()

    return k(table, gidx)


def _sc_segment_sum(e_rows, dst2, spw):
    @functools.partial(
        pl.kernel,
        out_type=jax.ShapeDtypeStruct((2, N, 128), jnp.float32),
        mesh=_mesh,
        scratch_types=[
            pltpu.VMEM((spw, CH), jnp.int32),
            pltpu.VMEM((SNBUF, CH, 128), jnp.float32),
            pltpu.VMEM((ZCH, 128), jnp.float32),
            pltpu.VMEM_SHARED((N + 8, 128), jnp.float32),
        ] + [pltpu.SemaphoreType.DMA] * SNBUF,
    )
    def k(e_hbm, idx_hbm, p_hbm, idx_v, rows_v, zbuf, acc, *sems):
        cid = lax.axis_index("c")
        sid = lax.axis_index("s")
        w = cid * 16 + sid

        @pl.loop(0, ZCH)
        def _(i):
            @pl.loop(0, 8)
            def _(l):
                zbuf[i, pl.ds(l * 16, 16)] = jnp.zeros((16,), jnp.float32)

        @pl.loop(0, NZCH // 16 + 1)
        def _(i):
            c = i * 16 + sid

            @pl.when(c < NZCH)
            def _():
                pltpu.sync_copy(zbuf, acc.at[pl.ds(c * ZCH, ZCH)])

        plsc.subcore_barrier()

        pltpu.sync_copy(idx_hbm.at[w], idx_v)

        @pl.loop(0, spw, step=SNBUF)
        def _(j):
            hl = [pltpu.async_copy(
                      e_hbm.at[pl.ds((w * spw + j + b) * CH, CH)],
                      rows_v.at[b], sems[b])
                  for b in range(SNBUF)]
            for b in range(SNBUF):
                hl[b].wait()
                pltpu.sync_copy(rows_v.at[b], acc.at[idx_v.at[j + b]],
                                add=True)

        plsc.subcore_barrier()

        @pl.loop(0, NZCH // 16 + 1)
        def _(i):
            c = i * 16 + sid

            @pl.when(c < NZCH)
            def _():
                pltpu.sync_copy(acc.at[pl.ds(c * ZCH, ZCH)],
                                p_hbm.at[cid, pl.ds(c * ZCH, ZCH)])

    return k(e_rows, dst2)


def _dot(x, w):
    return jnp.dot(x, w, preferred_element_type=jnp.float32)


def _wspec(shape):
    return pl.BlockSpec(shape, lambda i: (0,) * len(shape))


def _mlp3_body(x_ref, w1, b1, w2, b2, w3, b3, o_ref):
    h = jax.nn.relu(_dot(x_ref[...], w1[...]) + b1[...])
    h = jax.nn.relu(_dot(h, w2[...]) + b2[...])
    o_ref[...] = _dot(h, w3[...]) + b3[...]


def _mlp3(x, p, bm, out_rows=None):
    (w1, b1), (w2, b2), (w3, b3) = p
    m, din = x.shape
    dout = w3.shape[1]
    return pl.pallas_call(
        _mlp3_body,
        grid=(m // bm,),
        in_specs=[
            pl.BlockSpec((bm, din), lambda i: (i, 0)),
            _wspec(w1.shape), _wspec((1, w1.shape[1])),
            _wspec(w2.shape), _wspec((1, w2.shape[1])),
            _wspec(w3.shape), _wspec((1, w3.shape[1])),
        ],
        out_specs=pl.BlockSpec((bm, dout), lambda i: (i, 0)),
        out_shape=jax.ShapeDtypeStruct((out_rows or m, dout), jnp.float32),
    )(x, w1, b1.reshape(1, -1), w2, b2.reshape(1, -1), w3, b3.reshape(1, -1))


def _edge_mlp_body(e_ref, gs_ref, gd_ref, w1e, w1a, w1b, b1, w2, b2, w3, b3,
                   o_ref):
    h = (_dot(e_ref[...], w1e[...]) + _dot(gs_ref[...], w1a[...])
         + _dot(gd_ref[...], w1b[...]))
    h = jax.nn.relu(h + b1[...])
    h = jax.nn.relu(_dot(h, w2[...]) + b2[...])
    h = _dot(h, w3[...]) + b3[...]
    o_ref[...] = (h + e_ref[...]) * 0.5


def _edge_mlp(e, g, p, bm, rows, out_pad, eoff=0):
    (w1, b1), (w2, b2), (w3, b3) = p
    w1e, w1a, w1b = w1[:128], w1[128:256], w1[256:]
    goff = rows // bm
    return pl.pallas_call(
        _edge_mlp_body,
        grid=(rows // bm,),
        in_specs=[
            pl.BlockSpec((bm, 128), lambda i: (i + eoff, 0)),
            pl.BlockSpec((bm, 128), lambda i: (i, 0)),
            pl.BlockSpec((bm, 128), lambda i: (i + goff, 0)),
            _wspec((128, 256)), _wspec((128, 256)), _wspec((128, 256)),
            _wspec((1, 256)),
            _wspec((256, 256)), _wspec((1, 256)),
            _wspec((256, 128)), _wspec((1, 128)),
        ],
        out_specs=pl.BlockSpec((bm, 128), lambda i: (i, 0)),
        out_shape=jax.ShapeDtypeStruct((out_pad, 128), jnp.float32),
    )(e, g, g, w1e, w1a, w1b, b1.reshape(1, -1), w2, b2.reshape(1, -1), w3,
      b3.reshape(1, -1))


def _node_mlp_body(n_ref, pa_ref, pb_ref, w1n, w1s, b1, w2, b2, w3, b3,
                   o_ref):
    s = pa_ref[0] + pa_ref[1] + pb_ref[0] + pb_ref[1]
    h = _dot(n_ref[...], w1n[...]) + _dot(s, w1s[...])
    h = jax.nn.relu(h + b1[...])
    h = jax.nn.relu(_dot(h, w2[...]) + b2[...])
    h = _dot(h, w3[...]) + b3[...]
    o_ref[...] = (h + n_ref[...]) * 0.5


def _node_mlp(n_out, pa, pb, p, bm):
    (w1, b1), (w2, b2), (w3, b3) = p
    w1n, w1s = w1[:128], w1[128:]
    return pl.pallas_call(
        _node_mlp_body,
        grid=(N // bm,),
        in_specs=[
            pl.BlockSpec((bm, 128), lambda i: (i, 0)),
            pl.BlockSpec((2, bm, 128), lambda i: (0, i, 0)),
            pl.BlockSpec((2, bm, 128), lambda i: (0, i, 0)),
            _wspec((128, 256)), _wspec((128, 256)), _wspec((1, 256)),
            _wspec((256, 256)), _wspec((1, 256)),
            _wspec((256, 128)), _wspec((1, 128)),
        ],
        out_specs=pl.BlockSpec((bm, 128), lambda i: (i, 0)),
        out_shape=jax.ShapeDtypeStruct((N, 128), jnp.float32),
    )(n_out, pa, pb, w1n, w1s, b1.reshape(1, -1), w2, b2.reshape(1, -1),
      w3, b3.reshape(1, -1))


EH = E // 2
GPW_H = 40
SPW_H = 20
EPAD_H = NW * SPW_H * CH


def kernel(edges, n_feats, e_feats, n_feats_const, params):
    ed = edges[0]
    nf = jnp.concatenate([n_feats[0], n_feats_const[0]], axis=-1)
    ef = e_feats[0]

    gidx, dsts = [], []
    for h in range(2):
        edh = ed[h * EH:(h + 1) * EH]
        gp = (jnp.arange(NW * GPW_H * CH - 2 * EH, dtype=jnp.int32) * 1237) % N
        gidx.append(jnp.concatenate(
            [edh[:, 0], edh[:, 1], gp]).reshape(NW, GPW_H, CH))
        sp = TRASH + (jnp.arange(EPAD_H - EH, dtype=jnp.int32) % 8)
        dsts.append(jnp.concatenate([edh[:, 0], sp]).reshape(NW, SPW_H, CH))

    n_out = _mlp3(nf, params['n_encod'], 2000)
    eh = [_mlp3(ef[h * EH:(h + 1) * EH], params['e_encod'], 4000, EPAD_H)
          for h in range(2)]

    for e_p, n_p in zip(params['e_proc'], params['n_proc']):
        g = [_sc_gather(n_out, gidx[h], GPW_H) for h in range(2)]
        eh = [_edge_mlp(eh[h], g[h], e_p, 4000, EH, EPAD_H)
              for h in range(2)]
        pa = _sc_segment_sum(eh[0], dsts[0], SPW_H)
        pb = _sc_segment_sum(eh[1], dsts[1], SPW_H)
        n_out = _node_mlp(n_out, pa, pb, n_p, 2000)

    out = _mlp3(n_out, params['decod'], 2000)
    return out[None]


# speedup vs baseline: 1.1338x; 1.0960x over previous
import functools

import jax
import jax.numpy as jnp
from jax import lax
from jax.experimental import pallas as pl
from jax.experimental.pallas import tpu as pltpu
from jax.experimental.pallas import tpu_sc as plsc

N = 10000
E = 160000
NW = 32
CH = 128
NGCH = (2 * E) // CH
GPW = 80
GPAD = NW * GPW * CH
NSCH = E // CH
SPW = 40
EPAD = NW * SPW * CH
TRASH = N
ZCH = 40
NZCH = N // ZCH
NBUF = 4
SNBUF = 2

_mesh = plsc.VectorSubcoreMesh(core_axis_name="c", subcore_axis_name="s")


def _sc_gather(table, gidx, gpw):
    @functools.partial(
        pl.kernel,
        out_type=jax.ShapeDtypeStruct((NW * gpw * CH, 128), jnp.float32),
        mesh=_mesh,
        scratch_types=[
            pltpu.VMEM((gpw, CH), jnp.int32),
            pltpu.VMEM((NBUF, CH, 128), jnp.float32),
        ] + [pltpu.SemaphoreType.DMA] * (2 * NBUF),
    )
    def k(tab_hbm, idx_hbm, out_hbm, idx_v, rows_v, *sems):
        gsems, wsems = sems[:NBUF], sems[NBUF:]
        cid = lax.axis_index("c")
        sid = lax.axis_index("s")
        w = cid * 16 + sid
        pltpu.sync_copy(idx_hbm.at[w], idx_v)

        @pl.loop(0, gpw, step=NBUF)
        def _(j):
            hg = [pltpu.async_copy(tab_hbm.at[idx_v.at[j + b]],
                                   rows_v.at[b], gsems[b])
                  for b in range(NBUF)]
            hw = []
            for b in range(NBUF):
                hg[b].wait()
                c = (w * gpw + j + b) * CH
                hw.append(pltpu.async_copy(rows_v.at[b],
                                           out_hbm.at[pl.ds(c, CH)], wsems[b]))
            for b in range(NBUF):
                hw[b].wait()

    return k(table, gidx)


def _sc_segment_sum(e_rows, dst2, spw):
    @functools.partial(
        pl.kernel,
        out_type=jax.ShapeDtypeStruct((2, N, 128), jnp.float32),
        mesh=_mesh,
        scratch_types=[
            pltpu.VMEM((spw, CH), jnp.int32),
            pltpu.VMEM((SNBUF, CH, 128), jnp.float32),
            pltpu.VMEM((ZCH, 128), jnp.float32),
            pltpu.VMEM_SHARED((N + 8, 128), jnp.float32),
        ] + [pltpu.SemaphoreType.DMA] * SNBUF,
    )
    def k(e_hbm, idx_hbm, p_hbm, idx_v, rows_v, zbuf, acc, *sems):
        cid = lax.axis_index("c")
        sid = lax.axis_index("s")
        w = cid * 16 + sid

        @pl.loop(0, ZCH)
        def _(i):
            @pl.loop(0, 8)
            def _(l):
                zbuf[i, pl.ds(l * 16, 16)] = jnp.zeros((16,), jnp.float32)

        @pl.loop(0, NZCH // 16 + 1)
        def _(i):
            c = i * 16 + sid

            @pl.when(c < NZCH)
            def _():
                pltpu.sync_copy(zbuf, acc.at[pl.ds(c * ZCH, ZCH)])

        plsc.subcore_barrier()

        pltpu.sync_copy(idx_hbm.at[w], idx_v)

        @pl.loop(0, spw, step=SNBUF)
        def _(j):
            hl = [pltpu.async_copy(
                      e_hbm.at[pl.ds((w * spw + j + b) * CH, CH)],
                      rows_v.at[b], sems[b])
                  for b in range(SNBUF)]
            for b in range(SNBUF):
                hl[b].wait()
                pltpu.sync_copy(rows_v.at[b], acc.at[idx_v.at[j + b]],
                                add=True)

        plsc.subcore_barrier()

        @pl.loop(0, NZCH // 16 + 1)
        def _(i):
            c = i * 16 + sid

            @pl.when(c < NZCH)
            def _():
                pltpu.sync_copy(acc.at[pl.ds(c * ZCH, ZCH)],
                                p_hbm.at[cid, pl.ds(c * ZCH, ZCH)])

    return k(e_rows, dst2)


def _dot(x, w):
    return jnp.dot(x, w, preferred_element_type=jnp.float32)


def _wspec(shape):
    return pl.BlockSpec(shape, lambda i: (0,) * len(shape))


def _mlp3_body(x_ref, w1, b1, w2, b2, w3, b3, o_ref):
    h = jax.nn.relu(_dot(x_ref[...], w1[...]) + b1[...])
    h = jax.nn.relu(_dot(h, w2[...]) + b2[...])
    o_ref[...] = _dot(h, w3[...]) + b3[...]


def _mlp3(x, p, bm, out_rows=None):
    (w1, b1), (w2, b2), (w3, b3) = p
    m, din = x.shape
    dout = w3.shape[1]
    return pl.pallas_call(
        _mlp3_body,
        grid=(m // bm,),
        in_specs=[
            pl.BlockSpec((bm, din), lambda i: (i, 0)),
            _wspec(w1.shape), _wspec((1, w1.shape[1])),
            _wspec(w2.shape), _wspec((1, w2.shape[1])),
            _wspec(w3.shape), _wspec((1, w3.shape[1])),
        ],
        out_specs=pl.BlockSpec((bm, dout), lambda i: (i, 0)),
        out_shape=jax.ShapeDtypeStruct((out_rows or m, dout), jnp.float32),
    )(x, w1, b1.reshape(1, -1), w2, b2.reshape(1, -1), w3, b3.reshape(1, -1))


def _edge_mlp_body(e_ref, gs_ref, gd_ref, w1e, w1a, w1b, b1, w2, b2, w3, b3,
                   o_ref):
    h = (_dot(e_ref[...], w1e[...]) + _dot(gs_ref[...], w1a[...])
         + _dot(gd_ref[...], w1b[...]))
    h = jax.nn.relu(h + b1[...])
    h = jax.nn.relu(_dot(h, w2[...]) + b2[...])
    h = _dot(h, w3[...]) + b3[...]
    o_ref[...] = (h + e_ref[...]) * 0.5


def _edge_mlp(e, g, p, bm, rows, out_pad, eoff=0):
    (w1, b1), (w2, b2), (w3, b3) = p
    w1e, w1a, w1b = w1[:128], w1[128:256], w1[256:]
    goff = rows // bm
    return pl.pallas_call(
        _edge_mlp_body,
        grid=(rows // bm,),
        in_specs=[
            pl.BlockSpec((bm, 128), lambda i: (i + eoff, 0)),
            pl.BlockSpec((bm, 128), lambda i: (i, 0)),
            pl.BlockSpec((bm, 128), lambda i: (i + goff, 0)),
            _wspec((128, 256)), _wspec((128, 256)), _wspec((128, 256)),
            _wspec((1, 256)),
            _wspec((256, 256)), _wspec((1, 256)),
            _wspec((256, 128)), _wspec((1, 128)),
        ],
        out_specs=pl.BlockSpec((bm, 128), lambda i: (i, 0)),
        out_shape=jax.ShapeDtypeStruct((out_pad, 128), jnp.float32),
    )(e, g, g, w1e, w1a, w1b, b1.reshape(1, -1), w2, b2.reshape(1, -1), w3,
      b3.reshape(1, -1))


def _node_mlp_body(n_ref, pa_ref, pb_ref, w1n, w1s, b1, w2, b2, w3, b3,
                   o_ref):
    s = pa_ref[0] + pa_ref[1] + pb_ref[0] + pb_ref[1]
    h = _dot(n_ref[...], w1n[...]) + _dot(s, w1s[...])
    h = jax.nn.relu(h + b1[...])
    h = jax.nn.relu(_dot(h, w2[...]) + b2[...])
    h = _dot(h, w3[...]) + b3[...]
    o_ref[...] = (h + n_ref[...]) * 0.5


def _node_mlp(n_out, pa, pb, p, bm):
    (w1, b1), (w2, b2), (w3, b3) = p
    w1n, w1s = w1[:128], w1[128:]
    return pl.pallas_call(
        _node_mlp_body,
        grid=(N // bm,),
        in_specs=[
            pl.BlockSpec((bm, 128), lambda i: (i, 0)),
            pl.BlockSpec((2, bm, 128), lambda i: (0, i, 0)),
            pl.BlockSpec((2, bm, 128), lambda i: (0, i, 0)),
            _wspec((128, 256)), _wspec((128, 256)), _wspec((1, 256)),
            _wspec((256, 256)), _wspec((1, 256)),
            _wspec((256, 128)), _wspec((1, 128)),
        ],
        out_specs=pl.BlockSpec((bm, 128), lambda i: (i, 0)),
        out_shape=jax.ShapeDtypeStruct((N, 128), jnp.float32),
    )(n_out, pa, pb, w1n, w1s, b1.reshape(1, -1), w2, b2.reshape(1, -1),
      w3, b3.reshape(1, -1))


EH = E // 2
GPW_H = 40
SPW_H = 20
EPAD_H = NW * SPW_H * CH


def kernel(edges, n_feats, e_feats, n_feats_const, params):
    ed = edges[0]
    nf = jnp.concatenate([n_feats[0], n_feats_const[0]], axis=-1)
    ef = e_feats[0]

    gidx, dsts = [], []
    for h in range(2):
        edh = ed[h * EH:(h + 1) * EH]
        gp = (jnp.arange(NW * GPW_H * CH - 2 * EH, dtype=jnp.int32) * 1237) % N
        gidx.append(jnp.concatenate(
            [edh[:, 0], edh[:, 1], gp]).reshape(NW, GPW_H, CH))
        sp = TRASH + (jnp.arange(EPAD_H - EH, dtype=jnp.int32) % 8)
        dsts.append(jnp.concatenate([edh[:, 0], sp]).reshape(NW, SPW_H, CH))

    n_out = _mlp3(nf, params['n_encod'], 2000)
    e_full = _mlp3(ef, params['e_encod'], 4000, EPAD)
    eh = None

    for e_p, n_p in zip(params['e_proc'], params['n_proc']):
        g = [_sc_gather(n_out, gidx[h], GPW_H) for h in range(2)]
        if eh is None:
            eh = [_edge_mlp(e_full, g[h], e_p, 8000, EH, EPAD_H,
                            eoff=h * (EH // 8000))
                  for h in range(2)]
        else:
            eh = [_edge_mlp(eh[h], g[h], e_p, 8000, EH, EPAD_H)
                  for h in range(2)]
        pa = _sc_segment_sum(eh[0], dsts[0], SPW_H)
        pb = _sc_segment_sum(eh[1], dsts[1], SPW_H)
        n_out = _node_mlp(n_out, pa, pb, n_p, 2000)

    out = _mlp3(n_out, params['decod'], 2000)
    return out[None]
